# baseline (speedup 1.0000x reference)
import jax
import jax.numpy as jnp
from jax.experimental import pallas as pl
from jax.experimental.pallas import tpu as pltpu

B, S, D = 2, 2048, 1024
T = 16
L = 10
SBLK = 64


def _tc_body(scal_ref, x_ref, out_ref):
    z = scal_ref[0] * x_ref[0]
    sig = jax.nn.sigmoid(z)
    st = jnp.round(L * (1.0 - sig)).astype(jnp.int32)
    st = jnp.clip(st, 0, T - 1)
    tvals = jax.lax.broadcasted_iota(jnp.int32, (T, SBLK, D), 0)
    out_ref[0] = (tvals == st[None]).astype(jnp.float32)


def kernel(x, scaling):
    grid = (B, S // SBLK)
    return pl.pallas_call(
        _tc_body,
        grid=grid,
        in_specs=[
            pl.BlockSpec(memory_space=pltpu.SMEM),
            pl.BlockSpec((1, SBLK, D), lambda b, s: (b, s, 0)),
        ],
        out_specs=pl.BlockSpec((1, T, SBLK, D), lambda b, s: (b, 0, s, 0)),
        out_shape=jax.ShapeDtypeStruct((B, T, S, D), jnp.float32),
    )(scaling.reshape(1), x)


# speedup vs baseline: 1.1050x; 1.1050x over previous
import jax
import jax.numpy as jnp
from jax.experimental import pallas as pl
from jax.experimental.pallas import tpu as pltpu

B, S, D = 2, 2048, 1024
T = 16
L = 10
SBLK = 256


def _tc_body(scal_ref, x_ref, out_ref):
    z = scal_ref[0] * x_ref[0]
    sig = jax.nn.sigmoid(z)
    st = jnp.round(L * (1.0 - sig)).astype(jnp.int32)
    st = jnp.clip(st, 0, T - 1)
    tvals = jax.lax.broadcasted_iota(jnp.int32, (T, SBLK, D), 0)
    out_ref[0] = (tvals == st[None]).astype(jnp.float32)


def kernel(x, scaling):
    grid = (B, S // SBLK)
    return pl.pallas_call(
        _tc_body,
        grid=grid,
        in_specs=[
            pl.BlockSpec(memory_space=pltpu.SMEM),
            pl.BlockSpec((1, SBLK, D), lambda b, s: (b, s, 0)),
        ],
        out_specs=pl.BlockSpec((1, T, SBLK, D), lambda b, s: (b, 0, s, 0)),
        out_shape=jax.ShapeDtypeStruct((B, T, S, D), jnp.float32),
    )(scaling.reshape(1), x)


# speedup vs baseline: 1.1146x; 1.0087x over previous
import jax
import jax.numpy as jnp
from jax.experimental import pallas as pl
from jax.experimental.pallas import tpu as pltpu

B, S, D = 2, 2048, 1024
T = 16
L = 10
SBLK = 128


def _tc_body(scal_ref, x_ref, out_ref):
    z = scal_ref[0] * x_ref[0]
    sig = jax.nn.sigmoid(z)
    st = jnp.round(L * (1.0 - sig)).astype(jnp.int32)
    st = jnp.clip(st, 0, T - 1)
    tvals = jax.lax.broadcasted_iota(jnp.int32, (T, SBLK, D), 0)
    out_ref[0] = (tvals == st[None]).astype(jnp.float32)


def kernel(x, scaling):
    grid = (B, S // SBLK)
    return pl.pallas_call(
        _tc_body,
        grid=grid,
        in_specs=[
            pl.BlockSpec(memory_space=pltpu.SMEM),
            pl.BlockSpec((1, SBLK, D), lambda b, s: (b, s, 0)),
        ],
        out_specs=pl.BlockSpec((1, T, SBLK, D), lambda b, s: (b, 0, s, 0)),
        out_shape=jax.ShapeDtypeStruct((B, T, S, D), jnp.float32),
    )(scaling.reshape(1), x)
